# baseline (speedup 1.0000x reference)
import functools

import jax
import jax.numpy as jnp
import numpy as np
from jax.experimental import pallas as pl
from jax.experimental.pallas import tpu as pltpu
from jax.sharding import Mesh, PartitionSpec as P

_N = 10000
_IN_F = 128
_MID = 32
_OUT_F = 128
_BM = 200
_NBUF = 4


def _gcn_kernel(nsteps, x_ref, w1_ref, w2_ref, adj_ref, out_ref,
                abuf, ovmem, hid, in_sems, out_sem):
    for b in range(_NBUF):
        pltpu.make_async_copy(
            adj_ref.at[pl.ds(b * _BM, _BM), :], abuf.at[b], in_sems.at[b]
        ).start()

    hid[...] = jnp.dot(
        x_ref[...].astype(jnp.bfloat16),
        w1_ref[...].astype(jnp.bfloat16),
        preferred_element_type=jnp.float32,
    ).astype(jnp.bfloat16)

    def step(i, carry):
        slot = jax.lax.rem(i, _NBUF)
        row = i * _BM
        pltpu.make_async_copy(
            adj_ref.at[pl.ds(row, _BM), :], abuf.at[slot], in_sems.at[slot]
        ).wait()
        t = jnp.dot(
            abuf[slot].astype(jnp.bfloat16),
            hid[...],
            preferred_element_type=jnp.float32,
        )
        ovmem[pl.ds(row, _BM), :] = jnp.dot(
            t.astype(jnp.bfloat16),
            w2_ref[...].astype(jnp.bfloat16),
            preferred_element_type=jnp.float32,
        )
        pltpu.make_async_copy(
            ovmem.at[pl.ds(row, _BM), :], out_ref.at[pl.ds(row, _BM), :], out_sem
        ).start()
        nxt = i + _NBUF
        @pl.when(nxt < nsteps)
        def _():
            pltpu.make_async_copy(
                adj_ref.at[pl.ds(nxt * _BM, _BM), :], abuf.at[slot], in_sems.at[slot]
            ).start()
        return carry

    jax.lax.fori_loop(0, nsteps, step, 0)

    def drain(i, carry):
        row = i * _BM
        pltpu.make_async_copy(
            ovmem.at[pl.ds(row, _BM), :], out_ref.at[pl.ds(row, _BM), :], out_sem
        ).wait()
        return carry

    jax.lax.fori_loop(0, nsteps, drain, 0)


def _local_gcn(input, adj, weight1, weight2):
    rows = adj.shape[0]
    nsteps = rows // _BM
    return pl.pallas_call(
        functools.partial(_gcn_kernel, nsteps),
        in_specs=[
            pl.BlockSpec(memory_space=pltpu.MemorySpace.VMEM),
            pl.BlockSpec(memory_space=pltpu.MemorySpace.VMEM),
            pl.BlockSpec(memory_space=pltpu.MemorySpace.VMEM),
            pl.BlockSpec(memory_space=pltpu.MemorySpace.HBM),
        ],
        out_specs=pl.BlockSpec(memory_space=pltpu.MemorySpace.HBM),
        out_shape=jax.ShapeDtypeStruct((rows, _OUT_F), jnp.float32),
        scratch_shapes=[
            pltpu.VMEM((_NBUF, _BM, _N), jnp.float32),
            pltpu.VMEM((rows, _OUT_F), jnp.float32),
            pltpu.VMEM((_N, _MID), jnp.bfloat16),
            pltpu.SemaphoreType.DMA((_NBUF,)),
            pltpu.SemaphoreType.DMA,
        ],
    )(input, weight1, weight2, adj)


def kernel(input, adj, weight1, weight2):
    devs = jax.devices()
    nd = len(devs)
    if nd < 2 or _N % (nd * _BM) != 0:
        return _local_gcn(input, adj, weight1, weight2)
    mesh = Mesh(np.array(devs), ("i",))
    sharded = jax.shard_map(
        _local_gcn,
        mesh=mesh,
        in_specs=(P(None, None), P("i", None), P(None, None), P(None, None)),
        out_specs=P("i", None),
        check_vma=False,
    )
    return sharded(input, adj, weight1, weight2)


# speedup vs baseline: 5.3945x; 5.3945x over previous
import functools

import jax
import jax.numpy as jnp
from jax.experimental import pallas as pl
from jax.experimental.pallas import tpu as pltpu

_N = 10000
_IN_F = 128
_MID = 32
_OUT_F = 128
_BM = 200
_NBUF = 4
_NSTEPS = _N // _BM


def _gcn_kernel(x_ref, w1_ref, w2_ref, adj_ref, out_ref,
                abuf, ovmem, hid, in_sems, out_sem):
    for b in range(_NBUF):
        pltpu.make_async_copy(
            adj_ref.at[pl.ds(b * _BM, _BM), :], abuf.at[b], in_sems.at[b]
        ).start()

    hid[...] = jnp.dot(x_ref[...], w1_ref[...],
                       preferred_element_type=jnp.float32)

    def step(i, carry):
        slot = jax.lax.rem(i, _NBUF)
        row = i * _BM
        pltpu.make_async_copy(
            adj_ref.at[pl.ds(row, _BM), :], abuf.at[slot], in_sems.at[slot]
        ).wait()
        t = jnp.dot(abuf[slot], hid[...], preferred_element_type=jnp.float32)
        ovmem[pl.ds(row, _BM), :] = jnp.dot(
            t, w2_ref[...], preferred_element_type=jnp.float32)
        pltpu.make_async_copy(
            ovmem.at[pl.ds(row, _BM), :], out_ref.at[pl.ds(row, _BM), :], out_sem
        ).start()
        nxt = i + _NBUF
        @pl.when(nxt < _NSTEPS)
        def _():
            pltpu.make_async_copy(
                adj_ref.at[pl.ds(nxt * _BM, _BM), :], abuf.at[slot], in_sems.at[slot]
            ).start()
        return carry

    jax.lax.fori_loop(0, _NSTEPS, step, 0)

    def drain(i, carry):
        row = i * _BM
        pltpu.make_async_copy(
            ovmem.at[pl.ds(row, _BM), :], out_ref.at[pl.ds(row, _BM), :], out_sem
        ).wait()
        return carry

    jax.lax.fori_loop(0, _NSTEPS, drain, 0)


def kernel(input, adj, weight1, weight2):
    return pl.pallas_call(
        _gcn_kernel,
        in_specs=[
            pl.BlockSpec(memory_space=pltpu.MemorySpace.VMEM),
            pl.BlockSpec(memory_space=pltpu.MemorySpace.VMEM),
            pl.BlockSpec(memory_space=pltpu.MemorySpace.VMEM),
            pl.BlockSpec(memory_space=pltpu.MemorySpace.HBM),
        ],
        out_specs=pl.BlockSpec(memory_space=pltpu.MemorySpace.HBM),
        out_shape=jax.ShapeDtypeStruct((_N, _OUT_F), jnp.float32),
        scratch_shapes=[
            pltpu.VMEM((_NBUF, _BM, _N), jnp.float32),
            pltpu.VMEM((_N, _OUT_F), jnp.float32),
            pltpu.VMEM((_N, _MID), jnp.float32),
            pltpu.SemaphoreType.DMA((_NBUF,)),
            pltpu.SemaphoreType.DMA,
        ],
    )(input, weight1, weight2, adj)


# speedup vs baseline: 5.4360x; 1.0077x over previous
import jax
import jax.numpy as jnp
from jax.experimental import pallas as pl
from jax.experimental.pallas import tpu as pltpu

_N = 10000
_IN_F = 128
_MID = 32
_OUT_F = 128
_BM = 200
_NBUF = 4
_NSTEPS = _N // _BM


def _gcn_kernel(x_ref, w1_ref, w2_ref, adj_ref, out_ref,
                abuf, ovmem, sup, in_sems, out_sem):
    for b in range(_NBUF):
        pltpu.make_async_copy(
            adj_ref.at[pl.ds(b * _BM, _BM), :], abuf.at[b], in_sems.at[b]
        ).start()

    hid = jnp.dot(
        x_ref[...].astype(jnp.bfloat16),
        w1_ref[...].astype(jnp.bfloat16),
        preferred_element_type=jnp.float32,
    )
    sup[...] = jnp.dot(
        hid.astype(jnp.bfloat16),
        w2_ref[...].astype(jnp.bfloat16),
        preferred_element_type=jnp.float32,
    ).astype(jnp.bfloat16)

    def step(i, carry):
        slot = jax.lax.rem(i, _NBUF)
        row = i * _BM
        pltpu.make_async_copy(
            adj_ref.at[pl.ds(row, _BM), :], abuf.at[slot], in_sems.at[slot]
        ).wait()
        ovmem[pl.ds(row, _BM), :] = jnp.dot(
            abuf[slot].astype(jnp.bfloat16),
            sup[...],
            preferred_element_type=jnp.float32,
        )
        pltpu.make_async_copy(
            ovmem.at[pl.ds(row, _BM), :], out_ref.at[pl.ds(row, _BM), :], out_sem
        ).start()
        nxt = i + _NBUF
        @pl.when(nxt < _NSTEPS)
        def _():
            pltpu.make_async_copy(
                adj_ref.at[pl.ds(nxt * _BM, _BM), :], abuf.at[slot], in_sems.at[slot]
            ).start()
        return carry

    jax.lax.fori_loop(0, _NSTEPS, step, 0)

    def drain(i, carry):
        row = i * _BM
        pltpu.make_async_copy(
            ovmem.at[pl.ds(row, _BM), :], out_ref.at[pl.ds(row, _BM), :], out_sem
        ).wait()
        return carry

    jax.lax.fori_loop(0, _NSTEPS, drain, 0)


def kernel(input, adj, weight1, weight2):
    return pl.pallas_call(
        _gcn_kernel,
        in_specs=[
            pl.BlockSpec(memory_space=pltpu.MemorySpace.VMEM),
            pl.BlockSpec(memory_space=pltpu.MemorySpace.VMEM),
            pl.BlockSpec(memory_space=pltpu.MemorySpace.VMEM),
            pl.BlockSpec(memory_space=pltpu.MemorySpace.HBM),
        ],
        out_specs=pl.BlockSpec(memory_space=pltpu.MemorySpace.HBM),
        out_shape=jax.ShapeDtypeStruct((_N, _OUT_F), jnp.float32),
        scratch_shapes=[
            pltpu.VMEM((_NBUF, _BM, _N), jnp.float32),
            pltpu.VMEM((_N, _OUT_F), jnp.float32),
            pltpu.VMEM((_N, _OUT_F), jnp.bfloat16),
            pltpu.SemaphoreType.DMA((_NBUF,)),
            pltpu.SemaphoreType.DMA,
        ],
    )(input, weight1, weight2, adj)
